# baseline (speedup 1.0000x reference)
import jax
import jax.numpy as jnp
from jax.experimental import pallas as pl


def _void_fill_body(pix_ref, tex_ref, exp_ref, out_ref):
    mask = (pix_ref[...] >= 0).astype(jnp.bfloat16)
    mask4 = jax.lax.dot_general(
        mask, exp_ref[...],
        dimension_numbers=(((1,), (0,)), ((), ())),
        preferred_element_type=jnp.float32,
    )
    out_ref[...] = tex_ref[...] * mask4


def kernel(texels, pix_to_face):
    B, H, W, K, C = texels.shape
    rows = B * H * K
    tex2 = texels.reshape(rows, W * C)
    pix2 = pix_to_face.reshape(rows, W)
    expand = (jax.lax.broadcasted_iota(jnp.int32, (W, W * C), 1) // C
              == jax.lax.broadcasted_iota(jnp.int32, (W, W * C), 0)
              ).astype(jnp.bfloat16)
    R = 256
    out = pl.pallas_call(
        _void_fill_body,
        grid=(rows // R,),
        in_specs=[
            pl.BlockSpec((R, W), lambda i: (i, 0)),
            pl.BlockSpec((R, W * C), lambda i: (i, 0)),
            pl.BlockSpec((W, W * C), lambda i: (0, 0)),
        ],
        out_specs=pl.BlockSpec((R, W * C), lambda i: (i, 0)),
        out_shape=jax.ShapeDtypeStruct((rows, W * C), texels.dtype),
    )(pix2, tex2, expand)
    return out.reshape(texels.shape)


# speedup vs baseline: 2.4195x; 2.4195x over previous
import functools

import jax
import jax.numpy as jnp
from jax import lax
from jax.experimental import pallas as pl
from jax.experimental.pallas import tpu as pltpu
from jax.experimental.pallas import tpu_sc as plsc

_NW = 32
_CHUNK_TEX = 64
_CHUNK_PIX = 16


def _sc_body(tex_hbm, pix_hbm, out_hbm, tex_v, pix_v):
    wid = lax.axis_index("s") * 2 + lax.axis_index("c")
    rows_w = tex_hbm.shape[0] // _NW
    n_chunks = rows_w // _CHUNK_TEX
    base_tex = wid * rows_w
    base_pix = wid * (rows_w // 4)

    def chunk(g, carry):
        r0 = base_tex + g * _CHUNK_TEX
        p0 = base_pix + g * _CHUNK_PIX
        pltpu.sync_copy(tex_hbm.at[pl.ds(r0, _CHUNK_TEX)], tex_v)
        pltpu.sync_copy(pix_hbm.at[pl.ds(p0, _CHUNK_PIX)], pix_v)
        for p in range(_CHUNK_PIX):
            for l0 in range(0, 128, 16):
                keep = pix_v[p, pl.ds(l0, 16)] >= 0
                for c in range(4):
                    r = 4 * p + c
                    tv = tex_v[r, pl.ds(l0, 16)]
                    tex_v[r, pl.ds(l0, 16)] = jnp.where(keep, tv,
                                                        jnp.float32(0.0))
        pltpu.sync_copy(tex_v, out_hbm.at[pl.ds(r0, _CHUNK_TEX)])
        return carry

    lax.fori_loop(0, n_chunks, chunk, 0)


def kernel(texels, pix_to_face):
    B, H, W, K, C = texels.shape
    L = 128
    T = W // L
    tex2 = (texels.reshape(B, H, T, L, K, C)
            .transpose(0, 1, 2, 4, 5, 3)
            .reshape(B * H * T * K * C, L))
    pix2 = pix_to_face.reshape(B * H * K * T, L)
    rows = tex2.shape[0]

    sc_call = functools.partial(
        pl.kernel,
        out_type=jax.ShapeDtypeStruct((rows, L), texels.dtype),
        mesh=plsc.VectorSubcoreMesh(core_axis_name="c", subcore_axis_name="s"),
        scratch_types=[
            pltpu.VMEM((_CHUNK_TEX, L), jnp.float32),
            pltpu.VMEM((_CHUNK_PIX, L), jnp.int32),
        ],
    )(_sc_body)
    out2 = sc_call(tex2, pix2)
    return (out2.reshape(B, H, T, K, C, L)
            .transpose(0, 1, 2, 5, 3, 4)
            .reshape(B, H, W, K, C))


# speedup vs baseline: 9.0834x; 3.7543x over previous
import jax
import jax.numpy as jnp
from jax.experimental import pallas as pl
from jax.experimental.pallas import tpu as pltpu


def _void_fill_body(pix_ref, tex_ref, out_ref):
    keepw = ~jax.lax.shift_right_arithmetic(pix_ref[...], 31)
    keep4 = pltpu.bitcast(keepw, jnp.int8).astype(jnp.int32)
    tex_i = pltpu.bitcast(tex_ref[...], jnp.int32)
    out_ref[...] = pltpu.bitcast(tex_i & keep4, jnp.float32)


def kernel(texels, pix_to_face):
    B, H, W, K, C = texels.shape
    L = 128
    T = W // L
    tex2 = (texels.reshape(B, H, T, L, K, C)
            .transpose(0, 1, 2, 4, 5, 3)
            .reshape(B * H * T * K * C, L))
    pix2 = pix_to_face.reshape(B * H * K * T, L)
    G = 1024
    rows = tex2.shape[0]
    out2 = pl.pallas_call(
        _void_fill_body,
        grid=(rows // (4 * C * G),),
        in_specs=[
            pl.BlockSpec((C * G, L), lambda i: (i, 0)),
            pl.BlockSpec((4 * C * G, L), lambda i: (i, 0)),
        ],
        out_specs=pl.BlockSpec((4 * C * G, L), lambda i: (i, 0)),
        out_shape=jax.ShapeDtypeStruct((rows, L), texels.dtype),
    )(pix2, tex2)
    return (out2.reshape(B, H, T, K, C, L)
            .transpose(0, 1, 2, 5, 3, 4)
            .reshape(B, H, W, K, C))
